# baseline (speedup 1.0000x reference)
import functools

import jax
import jax.numpy as jnp
from jax import lax
from jax.experimental import pallas as pl
from jax.experimental.pallas import tpu as pltpu
from jax.experimental.pallas import tpu_sc as plsc

_N = 10000
_E = 160000
_D = 128
_GW = 5 * _D
_TW = _GW + _D
_NC = 2
_NS = 16
_NW = _NC * _NS
_EPW = _E // _NW
_C = 16
_NCHUNK = _E // _C
_RPT = 624
_TAIL = _N - _NS * _RPT


def _proj_body(x_ref, c0_ref, wl_ref, wr_ref, b_ref, outl_ref, outr_ref):
    xb = x_ref[...]
    outl_ref[:, :_GW] = (
        jnp.dot(xb, wl_ref[...], preferred_element_type=jnp.float32) + b_ref[...]
    )
    outl_ref[:, _GW:] = c0_ref[...]
    outr_ref[:, :_GW] = jnp.dot(xb, wr_ref[...], preferred_element_type=jnp.float32)
    outr_ref[:, _GW:] = c0_ref[...]


def _make_tables(x, c0, w_l, w_r, b_cat):
    blk = 1000
    return pl.pallas_call(
        _proj_body,
        grid=(_N // blk,),
        in_specs=[
            pl.BlockSpec((blk, _D), lambda i: (i, 0)),
            pl.BlockSpec((blk, _D), lambda i: (i, 0)),
            pl.BlockSpec((_D, _GW), lambda i: (0, 0)),
            pl.BlockSpec((_D, _GW), lambda i: (0, 0)),
            pl.BlockSpec((1, _GW), lambda i: (0, 0)),
        ],
        out_specs=[
            pl.BlockSpec((blk, _TW), lambda i: (i, 0)),
            pl.BlockSpec((blk, _TW), lambda i: (i, 0)),
        ],
        out_shape=[
            jax.ShapeDtypeStruct((_N, _TW), jnp.float32),
            jax.ShapeDtypeStruct((_N, _TW), jnp.float32),
        ],
    )(x, c0, w_l, w_r, b_cat)


def _sig(v):
    return 1.0 / (1.0 + jnp.exp(-v))


def _tanh(v):
    return 2.0 / (1.0 + jnp.exp(-2.0 * v)) - 1.0


_mesh = plsc.VectorSubcoreMesh(core_axis_name="c", subcore_axis_name="s")


@functools.partial(
    pl.kernel,
    out_type=jax.ShapeDtypeStruct((_NC, _N, _D), jnp.float32),
    mesh=_mesh,
    scratch_types=[
        pltpu.VMEM((_C,), jnp.int32),
        pltpu.VMEM((_C,), jnp.int32),
        pltpu.VMEM((_C, _TW), jnp.float32),
        pltpu.VMEM((_C, _TW), jnp.float32),
        pltpu.VMEM((_C, _D), jnp.float32),
        pltpu.VMEM_SHARED((_N, _D), jnp.float32),
        pltpu.SemaphoreType.DMA,
        pltpu.SemaphoreType.DMA,
    ],
)
def _edge_kernel(tl, tr, src, dst, out, sidx, didx, buf_l, buf_r, hbuf,
                 accum, sem_l, sem_r):
    cid = lax.axis_index("c")
    sid = lax.axis_index("s")
    wid = sid * _NC + cid

    def zrow(r, _):
        def zlane(k, _):
            hbuf[r, pl.ds(k * 16, 16)] = jnp.zeros((16,), jnp.float32)
            return 0
        return lax.fori_loop(0, _D // 16, zlane, 0)
    lax.fori_loop(0, _C, zrow, 0)

    def zcp(j, _):
        pltpu.sync_copy(hbuf, accum.at[pl.ds(sid * _RPT + j * _C, _C)])
        return 0
    lax.fori_loop(0, _RPT // _C, zcp, 0)

    @pl.when(sid == 0)
    def _():
        pltpu.sync_copy(hbuf, accum.at[pl.ds(_NS * _RPT, _TAIL)])
    plsc.subcore_barrier()

    nchunk = jnp.where(wid < _NCHUNK % _NW, _NCHUNK // _NW + 1, _NCHUNK // _NW)

    def chunk(t, _):
        off = (wid + t * _NW) * _C
        pltpu.sync_copy(src.at[pl.ds(off, _C)], sidx)
        pltpu.sync_copy(dst.at[pl.ds(off, _C)], didx)
        cp_l = pltpu.async_copy(tl.at[sidx], buf_l, sem_l)
        cp_r = pltpu.async_copy(tr.at[didx], buf_r, sem_r)
        cp_l.wait()
        cp_r.wait()

        def edge(e, _):
            def lane(k, _):
                o1 = k * 16
                i_ = _sig(buf_l[e, pl.ds(o1, 16)] + buf_r[e, pl.ds(o1, 16)])
                o_ = _sig(buf_l[e, pl.ds(_D + o1, 16)]
                          + buf_r[e, pl.ds(_D + o1, 16)])
                u_ = _tanh(buf_l[e, pl.ds(2 * _D + o1, 16)]
                           + buf_r[e, pl.ds(2 * _D + o1, 16)])
                fl_ = _sig(buf_l[e, pl.ds(3 * _D + o1, 16)]
                           + buf_r[e, pl.ds(3 * _D + o1, 16)])
                fr_ = _sig(buf_l[e, pl.ds(4 * _D + o1, 16)]
                           + buf_r[e, pl.ds(4 * _D + o1, 16)])
                c_ = (i_ * u_
                      + fl_ * buf_l[e, pl.ds(_GW + o1, 16)]
                      + fr_ * buf_r[e, pl.ds(_GW + o1, 16)])
                hbuf[e, pl.ds(o1, 16)] = o_ * _tanh(c_)
                return 0
            return lax.fori_loop(0, _D // 16, lane, 0)
        lax.fori_loop(0, _C, edge, 0)

        pltpu.sync_copy(hbuf, accum.at[didx], add=True)
        return 0
    lax.fori_loop(0, nchunk, chunk, 0)

    plsc.subcore_barrier()
    pltpu.sync_copy(accum.at[pl.ds(sid * _RPT, _RPT)],
                    out.at[cid, pl.ds(sid * _RPT, _RPT)])

    @pl.when(sid == 0)
    def _():
        pltpu.sync_copy(accum.at[pl.ds(_NS * _RPT, _TAIL)],
                        out.at[cid, pl.ds(_NS * _RPT, _TAIL)])


def _add_body(a_ref, b_ref, o_ref):
    o_ref[...] = a_ref[...] + b_ref[...]


def _add_partials(pa, pb):
    blk = 2000
    return pl.pallas_call(
        _add_body,
        grid=(_N // blk,),
        in_specs=[
            pl.BlockSpec((blk, _D), lambda i: (i, 0)),
            pl.BlockSpec((blk, _D), lambda i: (i, 0)),
        ],
        out_specs=pl.BlockSpec((blk, _D), lambda i: (i, 0)),
        out_shape=jax.ShapeDtypeStruct((_N, _D), jnp.float32),
    )(pa, pb)


def kernel(x, c0, edge_index, W_i, W_o, W_u, W_fl, W_fr, b_i, b_o, b_u, b_f):
    w_l = jnp.concatenate(
        [W_i[:_D], W_o[:_D], W_u[:_D], W_fl[:_D], W_fr[:_D]], axis=1)
    w_r = jnp.concatenate(
        [W_i[_D:], W_o[_D:], W_u[_D:], W_fl[_D:], W_fr[_D:]], axis=1)
    b_cat = jnp.concatenate([b_i, b_o, b_u, b_f, b_f]).reshape(1, _GW)
    tbl_l, tbl_r = _make_tables(x, c0, w_l, w_r, b_cat)
    partials = _edge_kernel(tbl_l, tbl_r, edge_index[0], edge_index[1])
    return _add_partials(partials[0], partials[1])
